# baseline (speedup 1.0000x reference)
import jax
import jax.numpy as jnp
from jax import lax
from jax.experimental import pallas as pl
from jax.experimental.pallas import tpu as pltpu
from jax.experimental.pallas import tpu_sc as plsc

N = 10000
E = 320000
DIM = 128
C = 16
NC = 2
NS = 16
NT = NC * NS
NPAD = 10240
ET = E // NT
GROUPS = ET // 16
K = 128
ROUND = 8
RING = 2
NCH = 80
CAP = NCH * K
ZROW = N
DROW = 0
RBLK = 512
NBLK = NPAD // RBLK
RPT = NPAD // NS

_MESH = plsc.VectorSubcoreMesh(core_axis_name="c", subcore_axis_name="s",
                               num_cores=NC, num_subcores=NS)
_SC_PARAMS = pltpu.CompilerParams(needs_layout_passes=False)


def _edge_scan_body(ei0_hbm, ei1_hbm, assign_hbm,
                    deg_hbm, cc_hbm, cu_hbm, cv_hbm, cnt_hbm,
                    assign_v, e0_v, e1_v, cu_v, cv_v, deg_v, cc_v, cnt_v):
    cid = lax.axis_index("c")
    sid = lax.axis_index("s")
    wid = cid * NS + sid
    base = wid * ET
    pltpu.sync_copy(assign_hbm, assign_v)
    pltpu.sync_copy(ei0_hbm.at[pl.ds(base, ET)], e0_v)
    pltpu.sync_copy(ei1_hbm.at[pl.ds(base, ET)], e1_v)

    zero16 = jnp.zeros((16,), jnp.float32)

    def zero_body(i, _):
        deg_v[pl.ds(i * 16, 16)] = zero16
        return 0

    lax.fori_loop(0, NPAD // 16, zero_body, 0)
    cc_v[...] = zero16

    ones = jnp.ones((16,), jnp.float32)

    def body(g, off):
        u = e0_v[pl.ds(g * 16, 16)]
        v = e1_v[pl.ds(g * 16, 16)]
        a0 = plsc.load_gather(assign_v, [u])
        a1 = plsc.load_gather(assign_v, [v])
        same = a0 == a1
        plsc.addupdate_scatter(deg_v, [v], ones, mask=same)
        plsc.addupdate_scatter(cc_v, [a0], ones, mask=same)
        plsc.store_compressed(cu_v.at[pl.ds(off, 16)], u, mask=same)
        plsc.store_compressed(cv_v.at[pl.ds(off, 16)], v, mask=same)
        return off + jnp.sum(same.astype(jnp.int32))

    m = lax.fori_loop(0, GROUPS, body, 0)

    zrow = jnp.full((16,), ZROW, jnp.int32)
    drow = jnp.full((16,), DROW, jnp.int32)
    for j in range(8):
        cu_v[pl.ds(m + j * 16, 16)] = zrow
        cv_v[pl.ds(m + j * 16, 16)] = drow

    nch = (m + K - 1) // K
    cnt_v[...] = jnp.full((16,), nch, jnp.int32)

    pltpu.sync_copy(deg_v, deg_hbm.at[wid])
    pltpu.sync_copy(cc_v, cc_hbm.at[wid])
    pltpu.sync_copy(cu_v, cu_hbm.at[wid])
    pltpu.sync_copy(cv_v, cv_hbm.at[wid])
    pltpu.sync_copy(cnt_v, cnt_hbm.at[wid])


_edge_scan = pl.kernel(
    _edge_scan_body,
    out_type=(jax.ShapeDtypeStruct((NT, NPAD), jnp.float32),
              jax.ShapeDtypeStruct((NT, C), jnp.float32),
              jax.ShapeDtypeStruct((NT, CAP), jnp.int32),
              jax.ShapeDtypeStruct((NT, CAP), jnp.int32),
              jax.ShapeDtypeStruct((NT, 16), jnp.int32)),
    mesh=_MESH,
    scratch_types=[
        pltpu.VMEM((N,), jnp.int32),
        pltpu.VMEM((ET,), jnp.int32),
        pltpu.VMEM((ET,), jnp.int32),
        pltpu.VMEM((CAP,), jnp.int32),
        pltpu.VMEM((CAP,), jnp.int32),
        pltpu.VMEM((NPAD,), jnp.float32),
        pltpu.VMEM((C,), jnp.float32),
        pltpu.VMEM((16,), jnp.int32),
    ],
    compiler_params=_SC_PARAMS,
)


def _mm_body(x_ref, w_ref, degt_ref, y_ref, dinv_ref):
    i = pl.program_id(0)
    deg = jnp.sum(degt_ref[...], axis=1, keepdims=True) + 1.0
    dinv = lax.rsqrt(deg)
    row = i * RBLK + lax.broadcasted_iota(jnp.int32, (RBLK, 1), 0)
    xm = jnp.where(row < N, x_ref[...], 0.0)
    y_ref[...] = jnp.dot(xm, w_ref[...], preferred_element_type=jnp.float32) * dinv
    dinv_ref[...] = dinv


def _mm(x, w, degt):
    return pl.pallas_call(
        _mm_body,
        grid=(NBLK,),
        in_specs=[pl.BlockSpec((RBLK, DIM), lambda i: (i, 0)),
                  pl.BlockSpec((DIM, DIM), lambda i: (0, 0)),
                  pl.BlockSpec((RBLK, NT), lambda i: (i, 0))],
        out_specs=[pl.BlockSpec((RBLK, DIM), lambda i: (i, 0)),
                   pl.BlockSpec((RBLK, 1), lambda i: (i, 0))],
        out_shape=[jax.ShapeDtypeStruct((NPAD, DIM), jnp.float32),
                   jax.ShapeDtypeStruct((NPAD, 1), jnp.float32)],
    )(x, w, degt)


def _msg_body(cu_hbm, cv_hbm, cnt_hbm, y_hbm, z_hbm, acc_hbm,
              uidx_v, vidx_v, cnt_v, rows_v, acc_sh,
              sem0, sem1, sem2, sem3):
    cid = lax.axis_index("c")
    sid = lax.axis_index("s")
    wid = cid * NS + sid
    sems = [sem0, sem1, sem2, sem3]
    pltpu.sync_copy(cnt_hbm.at[wid], cnt_v)
    pltpu.sync_copy(z_hbm, acc_sh.at[pl.ds(sid * RPT, RPT)])
    plsc.subcore_barrier()

    nch = jnp.max(cnt_v[...])

    def fire(b, j, buf):
        @pl.when(b + j < nch)
        def _():
            pltpu.async_copy(y_hbm.at[uidx_v.at[pl.ds(j * K, K)]],
                             rows_v.at[buf], sems[buf])

    def roundfn(r, _):
        b = r * ROUND
        ba = pl.multiple_of(b, ROUND)
        bk = pl.multiple_of(b * K, ROUND * K)
        pltpu.sync_copy(cu_hbm.at[wid, pl.ds(bk, ROUND * K)], uidx_v)
        pltpu.sync_copy(cv_hbm.at[wid, pl.ds(ba, ROUND)], vidx_v)
        for j in range(RING - 1):
            fire(b, j, j)
        for j in range(ROUND):
            jn = j + RING - 1
            if jn < ROUND:
                fire(b, jn, jn % RING)

            @pl.when(b + j < nch)
            def _(j=j):
                pltpu.make_async_copy(y_hbm.at[uidx_v.at[pl.ds(j * K, K)]],
                                      rows_v.at[j % RING],
                                      sems[j % RING]).wait()
                pltpu.sync_copy(rows_v.at[j % RING], acc_sh.at[vidx_v.at[j]],
                                add=True)

        return 0

    lax.fori_loop(0, (nch + ROUND - 1) // ROUND, roundfn, 0)
    plsc.subcore_barrier()
    pltpu.sync_copy(acc_sh.at[pl.ds(sid * RPT, RPT)],
                    acc_hbm.at[cid, pl.ds(sid * RPT, RPT)])


_msg = pl.kernel(
    _msg_body,
    out_type=jax.ShapeDtypeStruct((NC, NPAD, DIM), jnp.float32),
    mesh=_MESH,
    scratch_types=[
        pltpu.VMEM((ROUND * K,), jnp.int32),
        pltpu.VMEM((ROUND, K), jnp.int32),
        pltpu.VMEM((16,), jnp.int32),
        pltpu.VMEM((RING, K, DIM), jnp.float32),
        pltpu.VMEM_SHARED((NPAD, DIM), jnp.float32),
        pltpu.SemaphoreType.DMA,
        pltpu.SemaphoreType.DMA,
        pltpu.SemaphoreType.DMA,
        pltpu.SemaphoreType.DMA,
    ],
    compiler_params=_SC_PARAMS,
)


def _final_body(acc_ref, y_ref, x_ref, dinv_ref, cc_ref, asg_ref, b_ref, o_ref):
    dinv = dinv_ref[...]
    out = (acc_ref[0] + acc_ref[1] + y_ref[...]) * dinv + b_ref[...]
    hedge = (jnp.sum(cc_ref[...], axis=0, keepdims=True) > 0.0
             ).astype(jnp.float32)
    onehot = (asg_ref[...] == lax.broadcasted_iota(jnp.int32, (1, C), 1)
              ).astype(jnp.float32)
    updf = jnp.sum(onehot * hedge, axis=1, keepdims=True)
    o_ref[...] = jnp.where(updf > 0.0, out, x_ref[...])


def _final(acc2, y, x, dinv_col, cc, asg_col, b2):
    return pl.pallas_call(
        _final_body,
        grid=(NBLK,),
        in_specs=[pl.BlockSpec((NC, RBLK, DIM), lambda i: (0, i, 0)),
                  pl.BlockSpec((RBLK, DIM), lambda i: (i, 0)),
                  pl.BlockSpec((RBLK, DIM), lambda i: (i, 0)),
                  pl.BlockSpec((RBLK, 1), lambda i: (i, 0)),
                  pl.BlockSpec((NT, C), lambda i: (0, 0)),
                  pl.BlockSpec((RBLK, 1), lambda i: (i, 0)),
                  pl.BlockSpec((1, DIM), lambda i: (0, 0))],
        out_specs=pl.BlockSpec((RBLK, DIM), lambda i: (i, 0)),
        out_shape=jax.ShapeDtypeStruct((N, DIM), jnp.float32),
    )(acc2, y, x, dinv_col, cc, asg_col, b2)


def kernel(X, assign, full_ei, W, b):
    assign = assign.astype(jnp.int32)
    ei0 = full_ei[0].astype(jnp.int32)
    ei1 = full_ei[1].astype(jnp.int32)

    deg32, cc32, cu, cv, cnt = _edge_scan(ei0, ei1, assign)
    y, dinv_col = _mm(X, W, deg32.T)

    zeros = jnp.zeros((RPT, DIM), jnp.float32)
    acc2 = _msg(cu, cv.reshape(NT, NCH, K), cnt, y, zeros)

    asg_col = assign.reshape(N, 1)
    return _final(acc2, y, X, dinv_col, cc32, asg_col, b.reshape(1, DIM))


# speedup vs baseline: 1.3050x; 1.3050x over previous
import jax
import jax.numpy as jnp
from jax import lax
from jax.experimental import pallas as pl
from jax.experimental.pallas import tpu as pltpu
from jax.experimental.pallas import tpu_sc as plsc

N = 10000
E = 320000
DIM = 128
C = 16
NC = 2
NS = 16
NT = NC * NS
NPAD = 10240
ET = E // NT
GROUPS = ET // 16
K = 40
ROUND = 16
RING = 8
NCH = 256
CAP = NCH * K
ZROW = N
DROW = 0
RBLK = 512
NBLK = NPAD // RBLK
RPT = NPAD // NS

_MESH = plsc.VectorSubcoreMesh(core_axis_name="c", subcore_axis_name="s",
                               num_cores=NC, num_subcores=NS)
_SC_PARAMS = pltpu.CompilerParams(needs_layout_passes=False)


def _edge_scan_body(ei0_hbm, ei1_hbm, assign_hbm,
                    deg_hbm, cc_hbm, cu_hbm, cv_hbm, cnt_hbm,
                    assign_v, e0_v, e1_v, cu_v, cv_v, deg_v, cc_v, cnt_v):
    cid = lax.axis_index("c")
    sid = lax.axis_index("s")
    wid = cid * NS + sid
    base = wid * ET
    pltpu.sync_copy(assign_hbm, assign_v)
    pltpu.sync_copy(ei0_hbm.at[pl.ds(base, ET)], e0_v)
    pltpu.sync_copy(ei1_hbm.at[pl.ds(base, ET)], e1_v)

    zero16 = jnp.zeros((16,), jnp.float32)

    def zero_body(i, _):
        deg_v[pl.ds(i * 16, 16)] = zero16
        return 0

    lax.fori_loop(0, NPAD // 16, zero_body, 0)
    cc_v[...] = zero16

    ones = jnp.ones((16,), jnp.float32)

    def body(g, off):
        u = e0_v[pl.ds(g * 16, 16)]
        v = e1_v[pl.ds(g * 16, 16)]
        a0 = plsc.load_gather(assign_v, [u])
        a1 = plsc.load_gather(assign_v, [v])
        same = a0 == a1
        plsc.addupdate_scatter(deg_v, [v], ones, mask=same)
        plsc.addupdate_scatter(cc_v, [a0], ones, mask=same)
        plsc.store_compressed(cu_v.at[pl.ds(off, 16)], u, mask=same)
        plsc.store_compressed(cv_v.at[pl.ds(off, 16)], v, mask=same)
        return off + jnp.sum(same.astype(jnp.int32))

    m = lax.fori_loop(0, GROUPS, body, 0)

    zrow = jnp.full((16,), ZROW, jnp.int32)
    drow = jnp.full((16,), DROW, jnp.int32)
    for j in range(8):
        cu_v[pl.ds(m + j * 16, 16)] = zrow
        cv_v[pl.ds(m + j * 16, 16)] = drow

    nch = (m + K - 1) // K
    cnt_v[...] = jnp.full((16,), nch, jnp.int32)

    pltpu.sync_copy(deg_v, deg_hbm.at[wid])
    pltpu.sync_copy(cc_v, cc_hbm.at[wid])
    pltpu.sync_copy(cu_v, cu_hbm.at[wid])
    pltpu.sync_copy(cv_v, cv_hbm.at[wid])
    pltpu.sync_copy(cnt_v, cnt_hbm.at[wid])


_edge_scan = pl.kernel(
    _edge_scan_body,
    out_type=(jax.ShapeDtypeStruct((NT, NPAD), jnp.float32),
              jax.ShapeDtypeStruct((NT, C), jnp.float32),
              jax.ShapeDtypeStruct((NT, CAP), jnp.int32),
              jax.ShapeDtypeStruct((NT, CAP), jnp.int32),
              jax.ShapeDtypeStruct((NT, 16), jnp.int32)),
    mesh=_MESH,
    scratch_types=[
        pltpu.VMEM((N,), jnp.int32),
        pltpu.VMEM((ET,), jnp.int32),
        pltpu.VMEM((ET,), jnp.int32),
        pltpu.VMEM((CAP,), jnp.int32),
        pltpu.VMEM((CAP,), jnp.int32),
        pltpu.VMEM((NPAD,), jnp.float32),
        pltpu.VMEM((C,), jnp.float32),
        pltpu.VMEM((16,), jnp.int32),
    ],
    compiler_params=_SC_PARAMS,
)


def _mm_body(x_ref, w_ref, degt_ref, y_ref, dinv_ref):
    i = pl.program_id(0)
    deg = jnp.sum(degt_ref[...], axis=1, keepdims=True) + 1.0
    dinv = lax.rsqrt(deg)
    row = i * RBLK + lax.broadcasted_iota(jnp.int32, (RBLK, 1), 0)
    xm = jnp.where(row < N, x_ref[...], 0.0)
    y_ref[...] = jnp.dot(xm, w_ref[...], preferred_element_type=jnp.float32) * dinv
    dinv_ref[...] = dinv


def _mm(x, w, degt):
    return pl.pallas_call(
        _mm_body,
        grid=(NBLK,),
        in_specs=[pl.BlockSpec((RBLK, DIM), lambda i: (i, 0)),
                  pl.BlockSpec((DIM, DIM), lambda i: (0, 0)),
                  pl.BlockSpec((RBLK, NT), lambda i: (i, 0))],
        out_specs=[pl.BlockSpec((RBLK, DIM), lambda i: (i, 0)),
                   pl.BlockSpec((RBLK, 1), lambda i: (i, 0))],
        out_shape=[jax.ShapeDtypeStruct((NPAD, DIM), jnp.float32),
                   jax.ShapeDtypeStruct((NPAD, 1), jnp.float32)],
    )(x, w, degt)


def _msg_body(cu_hbm, cv_hbm, cnt_hbm, y_hbm, z_hbm, acc_hbm,
              uidx_v, vidx_v, cnt_v, rows_v, acc_sh,
              sem0, sem1, sem2, sem3, sem4, sem5, sem6, sem7):
    cid = lax.axis_index("c")
    sid = lax.axis_index("s")
    wid = cid * NS + sid
    sems = [sem0, sem1, sem2, sem3, sem4, sem5, sem6, sem7]
    pltpu.sync_copy(cnt_hbm.at[wid], cnt_v)
    pltpu.sync_copy(z_hbm, acc_sh.at[pl.ds(sid * RPT, RPT)])
    plsc.subcore_barrier()

    nch = jnp.max(cnt_v[...])

    def fire(b, j, buf):
        @pl.when(b + j < nch)
        def _():
            pltpu.async_copy(y_hbm.at[uidx_v.at[pl.ds(j * K, K)]],
                             rows_v.at[buf], sems[buf])

    def roundfn(r, _):
        b = r * ROUND
        ba = pl.multiple_of(b, ROUND)
        bk = pl.multiple_of(b * K, ROUND * K)
        pltpu.sync_copy(cu_hbm.at[wid, pl.ds(bk, ROUND * K)], uidx_v)
        pltpu.sync_copy(cv_hbm.at[wid, pl.ds(ba, ROUND)], vidx_v)
        for j in range(RING - 1):
            fire(b, j, j)
        for j in range(ROUND):
            jn = j + RING - 1
            if jn < ROUND:
                fire(b, jn, jn % RING)

            @pl.when(b + j < nch)
            def _(j=j):
                pltpu.make_async_copy(y_hbm.at[uidx_v.at[pl.ds(j * K, K)]],
                                      rows_v.at[j % RING],
                                      sems[j % RING]).wait()
                pltpu.sync_copy(rows_v.at[j % RING], acc_sh.at[vidx_v.at[j]],
                                add=True)

        return 0

    lax.fori_loop(0, (nch + ROUND - 1) // ROUND, roundfn, 0)
    plsc.subcore_barrier()
    pltpu.sync_copy(acc_sh.at[pl.ds(sid * RPT, RPT)],
                    acc_hbm.at[cid, pl.ds(sid * RPT, RPT)])


_msg = pl.kernel(
    _msg_body,
    out_type=jax.ShapeDtypeStruct((NC, NPAD, DIM), jnp.float32),
    mesh=_MESH,
    scratch_types=[
        pltpu.VMEM((ROUND * K,), jnp.int32),
        pltpu.VMEM((ROUND, K), jnp.int32),
        pltpu.VMEM((16,), jnp.int32),
        pltpu.VMEM((RING, K, DIM), jnp.float32),
        pltpu.VMEM_SHARED((NPAD, DIM), jnp.float32),
        pltpu.SemaphoreType.DMA,
        pltpu.SemaphoreType.DMA,
        pltpu.SemaphoreType.DMA,
        pltpu.SemaphoreType.DMA,
        pltpu.SemaphoreType.DMA,
        pltpu.SemaphoreType.DMA,
        pltpu.SemaphoreType.DMA,
        pltpu.SemaphoreType.DMA,
    ],
    compiler_params=_SC_PARAMS,
)


def _final_body(acc_ref, y_ref, x_ref, dinv_ref, cc_ref, asg_ref, b_ref, o_ref):
    dinv = dinv_ref[...]
    out = (acc_ref[0] + acc_ref[1] + y_ref[...]) * dinv + b_ref[...]
    hedge = (jnp.sum(cc_ref[...], axis=0, keepdims=True) > 0.0
             ).astype(jnp.float32)
    onehot = (asg_ref[...] == lax.broadcasted_iota(jnp.int32, (1, C), 1)
              ).astype(jnp.float32)
    updf = jnp.sum(onehot * hedge, axis=1, keepdims=True)
    o_ref[...] = jnp.where(updf > 0.0, out, x_ref[...])


def _final(acc2, y, x, dinv_col, cc, asg_col, b2):
    return pl.pallas_call(
        _final_body,
        grid=(NBLK,),
        in_specs=[pl.BlockSpec((NC, RBLK, DIM), lambda i: (0, i, 0)),
                  pl.BlockSpec((RBLK, DIM), lambda i: (i, 0)),
                  pl.BlockSpec((RBLK, DIM), lambda i: (i, 0)),
                  pl.BlockSpec((RBLK, 1), lambda i: (i, 0)),
                  pl.BlockSpec((NT, C), lambda i: (0, 0)),
                  pl.BlockSpec((RBLK, 1), lambda i: (i, 0)),
                  pl.BlockSpec((1, DIM), lambda i: (0, 0))],
        out_specs=pl.BlockSpec((RBLK, DIM), lambda i: (i, 0)),
        out_shape=jax.ShapeDtypeStruct((N, DIM), jnp.float32),
    )(acc2, y, x, dinv_col, cc, asg_col, b2)


def kernel(X, assign, full_ei, W, b):
    assign = assign.astype(jnp.int32)
    ei0 = full_ei[0].astype(jnp.int32)
    ei1 = full_ei[1].astype(jnp.int32)

    deg32, cc32, cu, cv, cnt = _edge_scan(ei0, ei1, assign)
    y, dinv_col = _mm(X, W, deg32.T)

    zeros = jnp.zeros((RPT, DIM), jnp.float32)
    acc2 = _msg(cu, cv.reshape(NT, NCH, K), cnt, y, zeros)

    asg_col = assign.reshape(N, 1)
    return _final(acc2, y, X, dinv_col, cc32, asg_col, b.reshape(1, DIM))
